# baseline (speedup 1.0000x reference)
import functools

import jax
import jax.numpy as jnp
from jax import lax
from jax.experimental import pallas as pl
from jax.experimental.pallas import tpu as pltpu
from jax.experimental.pallas import tpu_sc as plsc

_LANES = 16
_HC = 8


def _roll1(v, perm):
    return jnp.take_along_axis(v, perm, axis=0, mode="promise_in_bounds")


def _build_sc_kernel(C, H, Wl, Wr, D, interpret=False):
    mesh = plsc.VectorSubcoreMesh(
        core_axis_name="c", subcore_axis_name="s", num_cores=2, num_subcores=16
    )
    n_workers = mesh.num_cores * mesh.num_subcores
    assert C == n_workers and H % _HC == 0

    groups = Wl // _LANES
    n_chunks = H // _HC
    rpc = _HC * Wr // Wl

    @functools.partial(
        pl.kernel,
        out_type=jax.ShapeDtypeStruct((C, D, H, Wl), jnp.float32),
        mesh=mesh,
        scratch_types=[
            pltpu.VMEM((H, Wl), jnp.float32),
            pltpu.VMEM((2, rpc, Wl), jnp.float32),
            pltpu.VMEM((2, D, _HC, Wl), jnp.float32),
            pltpu.SemaphoreType.DMA,
            pltpu.SemaphoreType.DMA,
            pltpu.SemaphoreType.DMA,
        ],
        compiler_params=pltpu.CompilerParams(needs_layout_passes=False, use_tc_tiling_on_sc=False),
        interpret=interpret,
    )
    def k(left_hbm, right_hbm, out_hbm, left_v, rstage, stage, sem0, sem1, semr):
        wid = lax.axis_index("s") * mesh.num_cores + lax.axis_index("c")
        pltpu.sync_copy(left_hbm.at[wid], left_v)
        pltpu.async_copy(right_hbm.at[wid, pl.ds(0, rpc)], rstage.at[0], semr)
        lane = lax.iota(jnp.int32, _LANES)
        perm = (lane + (_LANES - 1)) & (_LANES - 1)
        l0 = lane == 0
        base = [lane * 4 + j * _LANES * 4 for j in range(groups)]
        sems = (sem0, sem1)

        def compute_chunk(ci, buf):
            rbuf = rstage.at[buf]

            def h_body(hl, carry2):
                h = ci * _HC + hl
                rbase = hl * Wr
                lfts = [left_v[h, pl.ds(j * _LANES, _LANES)] for j in range(groups)]
                row0 = rstage[buf, hl * (Wr // Wl), pl.ds(0, _LANES)]
                bval = jnp.take_along_axis(
                    row0, lane * 0, axis=0, mode="promise_in_bounds"
                )
                for r in range(4):
                    flats = [
                        jnp.maximum(base[j] + (1 - r), 0) + rbase
                        for j in range(groups)
                    ]
                    cur = [
                        plsc.load_gather(
                            rbuf,
                            [lax.shift_right_logical(f, 7), f & (Wl - 1)],
                        )
                        for f in flats
                    ]
                    for m in range(D // 4):
                        if m > 0:
                            rolls = [_roll1(v, perm) for v in cur]
                            cur = [
                                jnp.where(l0, bval if j == 0 else rolls[j - 1], rolls[j])
                                for j in range(groups)
                            ]
                        d = 4 * m + r
                        for j in range(groups):
                            stage[buf, d, hl, pl.ds(j * _LANES, _LANES)] = (
                                lfts[j] - cur[j]
                            )
                return carry2

            lax.fori_loop(0, _HC, h_body, 0, unroll=1)

        def c2_body(cc, carry):
            for buf in range(2):
                ci = cc * 2 + buf
                @pl.when(cc > 0)
                def _():
                    pltpu.make_async_copy(
                        stage.at[buf],
                        out_hbm.at[wid, :, pl.ds(ci * _HC, _HC)],
                        sems[buf],
                    ).wait()

                pltpu.make_async_copy(
                    right_hbm.at[wid, pl.ds(ci * rpc, rpc)], rstage.at[buf], semr
                ).wait()

                @pl.when(ci + 1 < n_chunks)
                def _():
                    pltpu.async_copy(
                        right_hbm.at[wid, pl.ds((ci + 1) * rpc, rpc)],
                        rstage.at[1 - buf],
                        semr,
                    )

                compute_chunk(ci, buf)
                pltpu.async_copy(
                    stage.at[buf],
                    out_hbm.at[wid, :, pl.ds(ci * _HC, _HC)],
                    sems[buf],
                )
            return carry

        lax.fori_loop(0, n_chunks // 2, c2_body, 0, unroll=False)
        for buf in range(2):
            pltpu.make_async_copy(
                stage.at[buf],
                out_hbm.at[wid, :, pl.ds((n_chunks - 2 + buf) * _HC, _HC)],
                sems[buf],
            ).wait()

    return k


def kernel(left_feature, right_feature, max_disp):
    B, C, H, Wl = left_feature.shape
    Wr = right_feature.shape[3]
    D = 48
    left3 = left_feature.reshape(C, H, Wl)
    right3 = right_feature.reshape(C, H * Wr // Wl, Wl)
    k = _build_sc_kernel(C, H, Wl, Wr, D)
    out = k(left3, right3)
    return out.reshape(B, C, D, H, Wl)


# speedup vs baseline: 1.0017x; 1.0017x over previous
import functools

import jax
import jax.numpy as jnp
from jax import lax
from jax.experimental import pallas as pl
from jax.experimental.pallas import tpu as pltpu
from jax.experimental.pallas import tpu_sc as plsc

_LANES = 16
_HC = 8


def _roll1(v, perm):
    return jnp.take_along_axis(v, perm, axis=0, mode="promise_in_bounds")


def _build_sc_kernel(C, H, Wl, Wr, D, interpret=False):
    mesh = plsc.VectorSubcoreMesh(
        core_axis_name="c", subcore_axis_name="s", num_cores=2, num_subcores=16
    )
    n_workers = mesh.num_cores * mesh.num_subcores
    assert C == n_workers and H % _HC == 0

    groups = Wl // _LANES
    n_chunks = H // _HC
    rpc = _HC * Wr // Wl

    @functools.partial(
        pl.kernel,
        out_type=jax.ShapeDtypeStruct((C, D, H, Wl), jnp.float32),
        mesh=mesh,
        scratch_types=[
            pltpu.VMEM((H, Wl), jnp.float32),
            pltpu.VMEM((2, rpc, Wl), jnp.float32),
            pltpu.VMEM((2, D, _HC, Wl), jnp.float32),
            pltpu.SemaphoreType.DMA,
            pltpu.SemaphoreType.DMA,
            pltpu.SemaphoreType.DMA,
        ],
        compiler_params=pltpu.CompilerParams(needs_layout_passes=False),
        interpret=interpret,
    )
    def k(left_hbm, right_hbm, out_hbm, left_v, rstage, stage, sem0, sem1, semr):
        wid = lax.axis_index("s") * mesh.num_cores + lax.axis_index("c")
        pltpu.sync_copy(left_hbm.at[wid], left_v)
        pltpu.async_copy(right_hbm.at[wid, pl.ds(0, rpc)], rstage.at[0], semr)
        lane = lax.iota(jnp.int32, _LANES)
        perm = (lane + (_LANES - 1)) & (_LANES - 1)
        l0 = lane == 0
        base = [lane * 4 + j * _LANES * 4 for j in range(groups)]
        sems = (sem0, sem1)

        def compute_chunk(ci, buf):
            rbuf = rstage.at[buf]

            def h_body(hl, carry2):
                h = ci * _HC + hl
                rbase = hl * Wr
                lfts = [left_v[h, pl.ds(j * _LANES, _LANES)] for j in range(groups)]
                row0 = rstage[buf, hl * (Wr // Wl), pl.ds(0, _LANES)]
                bval = jnp.take_along_axis(
                    row0, lane * 0, axis=0, mode="promise_in_bounds"
                )
                for r in range(4):
                    flats = [
                        jnp.maximum(base[j] + (1 - r), 0) + rbase
                        for j in range(groups)
                    ]
                    cur = [
                        plsc.load_gather(
                            rbuf,
                            [lax.shift_right_logical(f, 7), f & (Wl - 1)],
                        )
                        for f in flats
                    ]
                    for m in range(D // 4):
                        if m > 0:
                            rolls = [_roll1(v, perm) for v in cur]
                            cur = [
                                jnp.where(l0, bval if j == 0 else rolls[j - 1], rolls[j])
                                for j in range(groups)
                            ]
                        d = 4 * m + r
                        for j in range(groups):
                            stage[buf, d, hl, pl.ds(j * _LANES, _LANES)] = (
                                lfts[j] - cur[j]
                            )
                return carry2

            lax.fori_loop(0, _HC, h_body, 0, unroll=1)

        def c2_body(cc, carry):
            for buf in range(2):
                ci = cc * 2 + buf
                @pl.when(cc > 0)
                def _():
                    pltpu.make_async_copy(
                        stage.at[buf],
                        out_hbm.at[wid, :, pl.ds(ci * _HC, _HC)],
                        sems[buf],
                    ).wait()

                pltpu.make_async_copy(
                    right_hbm.at[wid, pl.ds(ci * rpc, rpc)], rstage.at[buf], semr
                ).wait()

                @pl.when(ci + 1 < n_chunks)
                def _():
                    pltpu.async_copy(
                        right_hbm.at[wid, pl.ds((ci + 1) * rpc, rpc)],
                        rstage.at[1 - buf],
                        semr,
                    )

                compute_chunk(ci, buf)
                pltpu.async_copy(
                    stage.at[buf],
                    out_hbm.at[wid, :, pl.ds(ci * _HC, _HC)],
                    sems[buf],
                )
            return carry

        lax.fori_loop(0, n_chunks // 2, c2_body, 0, unroll=False)
        for buf in range(2):
            pltpu.make_async_copy(
                stage.at[buf],
                out_hbm.at[wid, :, pl.ds((n_chunks - 2 + buf) * _HC, _HC)],
                sems[buf],
            ).wait()

    return k


def kernel(left_feature, right_feature, max_disp):
    B, C, H, Wl = left_feature.shape
    Wr = right_feature.shape[3]
    D = 48
    left3 = left_feature.reshape(C, H, Wl)
    right3 = right_feature.reshape(C, H * Wr // Wl, Wl)
    k = _build_sc_kernel(C, H, Wl, Wr, D)
    out = k(left3, right3)
    return out.reshape(B, C, D, H, Wl)
